# baseline (speedup 1.0000x reference)
import functools
import math

import jax
import jax.numpy as jnp
from jax import lax
from jax.experimental import pallas as pl
from jax.experimental.pallas import tpu as pltpu
from jax.experimental.pallas import tpu_sc as plsc

D_MODEL = 64
SCALE = math.sqrt(D_MODEL)

NUM_CORES = 2
NUM_SUBCORES = 16
NW = NUM_CORES * NUM_SUBCORES
LANES = 16


@functools.lru_cache(maxsize=None)
def _make_lookup(B, CH):
    n_per = B // NW
    n_chunks = n_per // CH

    mesh = plsc.VectorSubcoreMesh(core_axis_name="c", subcore_axis_name="s")

    @functools.partial(
        pl.kernel,
        out_type=jax.ShapeDtypeStruct((B, D_MODEL), jnp.float32),
        mesh=mesh,
        scratch_types=[
            pltpu.VMEM((CH,), jnp.int32),
            pltpu.VMEM((CH,), jnp.int32),
            pltpu.VMEM((CH, 2 * D_MODEL), jnp.float32),
            pltpu.VMEM((CH, D_MODEL), jnp.float32),
            pltpu.SemaphoreType.DMA,
        ],
        compiler_params=pltpu.CompilerParams(needs_layout_passes=False),
    )
    def lookup(x_hbm, emb2_hbm, out_hbm, idx_v, idxh_v, rows_v, outb_v, sem):
        wid = lax.axis_index("s") * NUM_CORES + lax.axis_index("c")
        base = wid * n_per

        def chunk_body(g, carry):
            cbase = base + g * CH
            pltpu.sync_copy(x_hbm.at[pl.ds(cbase, CH)], idx_v)

            def half_body(i, c):
                sl = pl.ds(i * LANES, LANES)
                idxh_v[sl] = idx_v[sl] >> 1
                return c

            lax.fori_loop(0, CH // LANES, half_body, 0, unroll=4)
            pltpu.async_copy(emb2_hbm.at[idxh_v], rows_v, sem).wait()

            def grp_body(gi, c):
                r0 = gi * LANES
                rr = r0 + lax.iota(jnp.int32, LANES)
                pp = (idx_v[pl.ds(r0, LANES)] & 1) * D_MODEL

                def col_body(cc, c2):
                    vals = plsc.load_gather(rows_v, [rr, pp + cc])
                    plsc.store_scatter(
                        outb_v,
                        [rr, jnp.full((LANES,), cc, jnp.int32)],
                        vals * SCALE,
                    )
                    return c2

                lax.fori_loop(0, D_MODEL, col_body, 0, unroll=8)
                return c

            lax.fori_loop(0, CH // LANES, grp_body, 0)
            pltpu.sync_copy(outb_v, out_hbm.at[pl.ds(cbase, CH)])
            return carry

        lax.fori_loop(0, n_chunks, chunk_body, 0)

    return lookup


def kernel(x, emb):
    bsz, hist = x.shape
    B = bsz * hist
    xf = x.reshape(B).astype(jnp.int32)
    emb2 = emb.reshape(emb.shape[0] // 2, 2 * D_MODEL)
    out = _make_lookup(B, 400)(xf, emb2)
    return out.reshape(1, bsz, hist, D_MODEL)


# speedup vs baseline: 2.4597x; 2.4597x over previous
import functools
import math

import jax
import jax.numpy as jnp
from jax import lax
from jax.experimental import pallas as pl
from jax.experimental.pallas import tpu as pltpu
from jax.experimental.pallas import tpu_sc as plsc

D_MODEL = 64
SCALE = math.sqrt(D_MODEL)

NUM_CORES = 2
NUM_SUBCORES = 16
NW = NUM_CORES * NUM_SUBCORES
LANES = 16


@functools.lru_cache(maxsize=None)
def _make_lookup(B, CH):
    n_per = B // NW
    n_chunks = n_per // CH

    mesh = plsc.VectorSubcoreMesh(core_axis_name="c", subcore_axis_name="s")

    @functools.partial(
        pl.kernel,
        out_type=jax.ShapeDtypeStruct((B, D_MODEL), jnp.float32),
        mesh=mesh,
        scratch_types=[
            pltpu.VMEM((CH,), jnp.int32),
            pltpu.VMEM((CH, D_MODEL), jnp.float32),
            pltpu.SemaphoreType.DMA,
        ],
        compiler_params=pltpu.CompilerParams(needs_layout_passes=False),
    )
    def lookup(x_hbm, emb_hbm, out_hbm, idx_s, rows_v, sem):
        wid = lax.axis_index("s") * NUM_CORES + lax.axis_index("c")
        base = wid * n_per

        def chunk_body(g, carry):
            cbase = base + g * CH
            pltpu.sync_copy(x_hbm.at[pl.ds(cbase, CH)], idx_s)

            def row_fire(gi, c):
                r0 = gi * LANES
                vec = idx_s[pl.ds(r0, LANES)]
                for l in range(LANES):
                    pltpu.make_async_copy(
                        emb_hbm.at[vec[l]], rows_v.at[r0 + l], sem
                    ).start()
                return c

            lax.fori_loop(0, CH // LANES, row_fire, 0)
            pltpu.make_async_copy(
                emb_hbm.at[pl.ds(0, CH)], rows_v, sem
            ).wait()

            def scale_body(r, c):
                for j in range(D_MODEL // LANES):
                    sl = pl.ds(j * LANES, LANES)
                    rows_v[r, sl] = rows_v[r, sl] * SCALE
                return c

            lax.fori_loop(0, CH, scale_body, 0, unroll=4)
            pltpu.sync_copy(rows_v, out_hbm.at[pl.ds(cbase, CH)])
            return carry

        lax.fori_loop(0, n_chunks, chunk_body, 0)

    return lookup


def kernel(x, emb):
    bsz, hist = x.shape
    B = bsz * hist
    xf = x.reshape(B).astype(jnp.int32)
    out = _make_lookup(B, 400)(xf, emb)
    return out.reshape(1, bsz, hist, D_MODEL)
